# baseline (speedup 1.0000x reference)
import functools

import jax
import jax.numpy as jnp
from jax import lax
from jax.experimental import pallas as pl
from jax.experimental.pallas import tpu as pltpu
from jax.experimental.pallas import tpu_sc as plsc

N = 10000
E = 320000
NPAD = 10240
NC = 2
NS = 16
NW = NC * NS
EPW = E // NW
L = 16
K = 80
NSTAGES = EPW // K
SEG = NPAD // NS
D = 64
DP = D + 16
NEG = -1e30


def _mesh():
    return plsc.VectorSubcoreMesh(core_axis_name="c", subcore_axis_name="s")


def _pad_cols(h):
    n = h.shape[0]
    ones = jnp.ones((n, 1), jnp.float32)
    zeros = jnp.zeros((n, 15), jnp.float32)
    return jnp.concatenate([h, ones, zeros], axis=1)


def _tc_first_body(x_ref, w_ref, asrc_ref, adst_ref,
                   ha_ref, hb_ref, als_ref, ald_ref):
    h = jnp.dot(x_ref[...], w_ref[...], preferred_element_type=jnp.float32)
    ha_ref[...] = _pad_cols(h[:, :D])
    hb_ref[...] = _pad_cols(h[:, D:])
    als_ref[...] = h @ asrc_ref[...]
    ald_ref[...] = h @ adst_ref[...]


def _tc_first(x, W, a_src, a_dst):
    ha, hb, al_s, al_d = pl.pallas_call(
        _tc_first_body,
        out_shape=(
            jax.ShapeDtypeStruct((N, DP), jnp.float32),
            jax.ShapeDtypeStruct((N, DP), jnp.float32),
            jax.ShapeDtypeStruct((N, 1), jnp.float32),
            jax.ShapeDtypeStruct((N, 1), jnp.float32),
        ),
    )(x, W, a_src[:, None], a_dst[:, None])
    return ha, hb, al_s[:, 0], al_d[:, 0]


def _tc_mid_body(oa_ref, ob_ref, b_ref, w_ref, asrc_ref, adst_ref,
                 h_ref, als_ref, ald_ref):
    sa = oa_ref[0] + oa_ref[1]
    sb = ob_ref[0] + ob_ref[1]
    den = sa[:, D:D + 1] + 1e-16
    o = jnp.concatenate([sa[:, :D], sb[:, :D]], axis=1) / den + b_ref[...]
    x2 = jnp.where(o > 0, o, jnp.exp(jnp.minimum(o, 0.0)) - 1.0)
    h = jnp.dot(x2, w_ref[...], preferred_element_type=jnp.float32)
    h_ref[...] = _pad_cols(h)
    als_ref[...] = h @ asrc_ref[...]
    ald_ref[...] = h @ adst_ref[...]


def _tc_mid(oa, ob, b, W, a_src, a_dst):
    h_pad, al_s, al_d = pl.pallas_call(
        _tc_mid_body,
        out_shape=(
            jax.ShapeDtypeStruct((NPAD, DP), jnp.float32),
            jax.ShapeDtypeStruct((NPAD, 1), jnp.float32),
            jax.ShapeDtypeStruct((NPAD, 1), jnp.float32),
        ),
    )(oa, ob, b[None, :], W, a_src[:, None], a_dst[:, None])
    return h_pad, al_s[:, 0], al_d[:, 0]


def _tc_final_body(acc_ref, b_ref, out_ref):
    s = acc_ref[0] + acc_ref[1]
    o = s[:, :D] / (s[:, D:D + 1] + 1e-16) + b_ref[...]
    m = jnp.max(o, axis=1, keepdims=True)
    z = o - m
    out_ref[...] = z - jnp.log(jnp.sum(jnp.exp(z), axis=1, keepdims=True))


def _tc_final(acc, b):
    return pl.pallas_call(
        _tc_final_body,
        out_shape=jax.ShapeDtypeStruct((NPAD, D), jnp.float32),
    )(acc, b[None, :])


def _sc_edge_max(src, dst, asrc, adst):
    nt = asrc.shape[0]

    def body(src_h, dst_h, asrc_h, adst_h, e_h, m_h,
             asrc_v, adst_v, mx, src_c, dst_c, e_c, red_v, mseg, stage):
        cid = lax.axis_index("c")
        sid = lax.axis_index("s")
        wid = sid * NC + cid
        base = wid * EPW
        pltpu.sync_copy(asrc_h, asrc_v)
        pltpu.sync_copy(adst_h, adst_v)
        pltpu.sync_copy(src_h.at[pl.ds(base, EPW)], src_c)
        pltpu.sync_copy(dst_h.at[pl.ds(base, EPW)], dst_c)

        def zero_body(i, c):
            mx[pl.ds(i * L, L)] = jnp.full((L,), NEG, jnp.float32)
            return c
        lax.fori_loop(0, NPAD // L, zero_body, 0)

        def edge_body(j, c):
            s = src_c[pl.ds(j * L, L)]
            d = dst_c[pl.ds(j * L, L)]
            e = plsc.load_gather(asrc_v, [s]) + plsc.load_gather(adst_v, [d])
            e = jnp.where(e >= 0, e, 0.2 * e)
            e_c[pl.ds(j * L, L)] = e

            def cond(p):
                return jnp.max(jnp.where(p, 1, 0)) > 0

            def wbody(p):
                cur = plsc.load_gather(mx, [d])
                plsc.store_scatter(mx, [d], jnp.maximum(cur, e), mask=p)
                cur2 = plsc.load_gather(mx, [d])
                return jnp.logical_and(p, cur2 < e)

            lax.while_loop(cond, wbody, jnp.ones((L,), jnp.bool_))
            return c
        lax.fori_loop(0, EPW // L, edge_body, 0)
        pltpu.sync_copy(e_c, e_h.at[pl.ds(base, EPW)])

        pltpu.sync_copy(mx, stage.at[sid])
        plsc.subcore_barrier()
        pltpu.sync_copy(stage.at[:, pl.ds(sid * SEG, SEG)], red_v)

        def red_body(j, c):
            acc = red_v[0, pl.ds(j * L, L)]
            for t in range(1, NS):
                acc = jnp.maximum(acc, red_v[t, pl.ds(j * L, L)])
            mseg[pl.ds(j * L, L)] = acc
            return c
        lax.fori_loop(0, SEG // L, red_body, 0)
        pltpu.sync_copy(mseg, m_h.at[cid, pl.ds(sid * SEG, SEG)])

    fn = pl.kernel(
        body,
        out_type=(
            jax.ShapeDtypeStruct((E,), jnp.float32),
            jax.ShapeDtypeStruct((NC, NPAD), jnp.float32),
        ),
        mesh=_mesh(),
        scratch_types=[
            pltpu.VMEM((nt,), jnp.float32),
            pltpu.VMEM((nt,), jnp.float32),
            pltpu.VMEM((NPAD,), jnp.float32),
            pltpu.VMEM((EPW,), jnp.int32),
            pltpu.VMEM((EPW,), jnp.int32),
            pltpu.VMEM((EPW,), jnp.float32),
            pltpu.VMEM((NS, SEG), jnp.float32),
            pltpu.VMEM((SEG,), jnp.float32),
            pltpu.VMEM_SHARED((NS, NPAD), jnp.float32),
        ],
        name="sc_edge_max",
        compiler_params=pltpu.CompilerParams(needs_layout_passes=False),
    )
    return fn(src, dst, asrc, adst)


def _sc_aggregate(src, dst, e_edge, m2, h_pad):
    zeros_h = jnp.zeros((NPAD, DP), jnp.float32)

    def body(src_h, dst_h, e_h, m_h, h_h, z_h, out_h,
             m_v, mtmp, src_c, dst_c, e_c, sidx, didx, wv,
             rows0, rows1, out_sh, gsem0, gsem1, ssem0, ssem1):
        cid = lax.axis_index("c")
        sid = lax.axis_index("s")
        wid = sid * NC + cid
        base = wid * EPW
        rows = (rows0, rows1)
        gsems = (gsem0, gsem1)
        ssems = (ssem0, ssem1)

        pltpu.sync_copy(m_h.at[0], m_v)
        pltpu.sync_copy(src_h.at[pl.ds(base, EPW)], src_c)
        pltpu.sync_copy(dst_h.at[pl.ds(base, EPW)], dst_c)
        pltpu.sync_copy(e_h.at[pl.ds(base, EPW)], e_c)

        def max_chunk(k, c):
            pltpu.sync_copy(m_h.at[1, pl.ds(k * SEG, SEG)], mtmp)

            def max_body(j, cc):
                off = k * SEG + j * L
                m_v[pl.ds(off, L)] = jnp.maximum(
                    m_v[pl.ds(off, L)], mtmp[pl.ds(j * L, L)])
                return cc
            lax.fori_loop(0, SEG // L, max_body, 0)
            return c
        lax.fori_loop(0, NS, max_chunk, 0)

        pltpu.sync_copy(z_h.at[pl.ds(sid * SEG, SEG)],
                        out_sh.at[pl.ds(sid * SEG, SEG)])
        plsc.subcore_barrier()

        def prep(t, q):
            for j in range(K // L):
                off = t * K + j * L
                s = src_c[pl.ds(off, L)]
                dd = dst_c[pl.ds(off, L)]
                e = e_c[pl.ds(off, L)]
                mval = plsc.load_gather(m_v, [dd])
                w = jnp.exp(e - mval)
                sidx[q, pl.ds(j * L, L)] = s
                didx[q, pl.ds(j * L, L)] = dd
                wv[q, pl.ds(j * L, L)] = w

        def fire_gather(q):
            pltpu.async_copy(h_h.at[sidx.at[q]], rows[q], gsems[q])

        def wait_gather(q):
            pltpu.make_async_copy(h_h.at[sidx.at[q]], rows[q], gsems[q]).wait()

        def fire_scatter(q):
            pltpu.async_copy(rows[q], out_sh.at[didx.at[q]], ssems[q],
                             add=True)

        def wait_scatter(q):
            pltpu.make_async_copy(rows[q], out_sh.at[didx.at[q]],
                                  ssems[q]).wait()

        def scale(q):
            def scale_body(i, c):
                ridx = jnp.full((L,), 0, jnp.int32) + i
                wb = plsc.load_gather(wv.at[q], [ridx])
                for cc in range(DP // L):
                    cidx = lax.iota(jnp.int32, L) + cc * L
                    v = plsc.load_gather(rows[q], [ridx, cidx])
                    plsc.store_scatter(rows[q], [ridx, cidx], v * wb)
                return c
            lax.fori_loop(0, K, scale_body, 0)

        pltpu.sync_copy(z_h.at[pl.ds(0, K)], rows1)
        for j in range(K // L):
            didx[1, pl.ds(j * L, L)] = jnp.zeros((L,), jnp.int32)
        fire_scatter(1)

        prep(0, 0)
        fire_gather(0)

        def half(s, p):
            wait_scatter(1 - p)
            prep(s + 1, 1 - p)
            fire_gather(1 - p)
            wait_gather(p)
            scale(p)
            fire_scatter(p)

        def stage_pair(i2, c):
            half(2 * i2, 0)
            half(2 * i2 + 1, 1)
            return c
        lax.fori_loop(0, (NSTAGES - 1) // 2, stage_pair, 0)
        wait_gather(0)
        scale(0)
        fire_scatter(0)
        wait_scatter(1)
        wait_scatter(0)

        plsc.subcore_barrier()
        pltpu.sync_copy(out_sh.at[pl.ds(sid * SEG, SEG)],
                        out_h.at[cid, pl.ds(sid * SEG, SEG)])

    fn = pl.kernel(
        body,
        out_type=jax.ShapeDtypeStruct((NC, NPAD, DP), jnp.float32),
        mesh=_mesh(),
        scratch_types=[
            pltpu.VMEM((NPAD,), jnp.float32),
            pltpu.VMEM((SEG,), jnp.float32),
            pltpu.VMEM((EPW,), jnp.int32),
            pltpu.VMEM((EPW,), jnp.int32),
            pltpu.VMEM((EPW,), jnp.float32),
            pltpu.VMEM((2, K), jnp.int32),
            pltpu.VMEM((2, K), jnp.int32),
            pltpu.VMEM((2, K), jnp.float32),
            pltpu.VMEM((K, DP), jnp.float32),
            pltpu.VMEM((K, DP), jnp.float32),
            pltpu.VMEM_SHARED((NPAD, DP), jnp.float32),
            pltpu.SemaphoreType.DMA,
            pltpu.SemaphoreType.DMA,
            pltpu.SemaphoreType.DMA,
            pltpu.SemaphoreType.DMA,
        ],
        name="sc_aggregate",
        compiler_params=pltpu.CompilerParams(
            needs_layout_passes=False, use_tc_tiling_on_sc=False),
    )
    return fn(src, dst, e_edge, m2, h_pad, zeros_h)


def kernel(x, edge_index, W1, a1_src, a1_dst, b1, W2, a2_src, a2_dst, b2):
    src = edge_index[0]
    dst = edge_index[1]
    ha1, hb1, as1, ad1 = _tc_first(x, W1, a1_src, a1_dst)
    e1, m1 = _sc_edge_max(src, dst, as1, ad1)
    o1a = _sc_aggregate(src, dst, e1, m1, ha1)
    o1b = _sc_aggregate(src, dst, e1, m1, hb1)
    h2p, as2, ad2 = _tc_mid(o1a, o1b, b1, W2, a2_src, a2_dst)
    e2, m2 = _sc_edge_max(src, dst, as2, ad2)
    o2 = _sc_aggregate(src, dst, e2, m2, h2p)
    out = _tc_final(o2, b2)
    return out[:N]


# speedup vs baseline: 1.8722x; 1.8722x over previous
import functools

import jax
import jax.numpy as jnp
from jax import lax
from jax.experimental import pallas as pl
from jax.experimental.pallas import tpu as pltpu
from jax.experimental.pallas import tpu_sc as plsc

N = 10000
E = 320000
NPAD = 10240
NC = 2
NS = 16
NW = NC * NS
EPW = E // NW
L = 16
K = 80
NSTAGES = EPW // K
SEG = NPAD // NS
D = 64
DP = D + 16
NEG = -1e30


def _mesh():
    return plsc.VectorSubcoreMesh(core_axis_name="c", subcore_axis_name="s")


def _pad_cols(h):
    n = h.shape[0]
    ones = jnp.ones((n, 1), jnp.float32)
    zeros = jnp.zeros((n, 15), jnp.float32)
    return jnp.concatenate([h, ones, zeros], axis=1)


def _tc_first_body(x_ref, w_ref, asrc_ref, adst_ref,
                   ha_ref, hb_ref, als_ref, ald_ref):
    h = jnp.dot(x_ref[...], w_ref[...], preferred_element_type=jnp.float32)
    ha_ref[...] = _pad_cols(h[:, :D])
    hb_ref[...] = _pad_cols(h[:, D:])
    als_ref[...] = h @ asrc_ref[...]
    ald_ref[...] = h @ adst_ref[...]


def _tc_first(x, W, a_src, a_dst):
    ha, hb, al_s, al_d = pl.pallas_call(
        _tc_first_body,
        out_shape=(
            jax.ShapeDtypeStruct((N, DP), jnp.float32),
            jax.ShapeDtypeStruct((N, DP), jnp.float32),
            jax.ShapeDtypeStruct((N, 1), jnp.float32),
            jax.ShapeDtypeStruct((N, 1), jnp.float32),
        ),
    )(x, W, a_src[:, None], a_dst[:, None])
    return ha, hb, al_s[:, 0], al_d[:, 0]


def _tc_mid_body(oa_ref, ob_ref, b_ref, w_ref, asrc_ref, adst_ref,
                 h_ref, als_ref, ald_ref):
    sa = oa_ref[0] + oa_ref[1]
    sb = ob_ref[0] + ob_ref[1]
    den = sa[:, D:D + 1] + 1e-16
    o = jnp.concatenate([sa[:, :D], sb[:, :D]], axis=1) / den + b_ref[...]
    x2 = jnp.where(o > 0, o, jnp.exp(jnp.minimum(o, 0.0)) - 1.0)
    h = jnp.dot(x2, w_ref[...], preferred_element_type=jnp.float32)
    h_ref[...] = _pad_cols(h)
    als_ref[...] = h @ asrc_ref[...]
    ald_ref[...] = h @ adst_ref[...]


def _tc_mid(oa, ob, b, W, a_src, a_dst):
    h_pad, al_s, al_d = pl.pallas_call(
        _tc_mid_body,
        out_shape=(
            jax.ShapeDtypeStruct((NPAD, DP), jnp.float32),
            jax.ShapeDtypeStruct((NPAD, 1), jnp.float32),
            jax.ShapeDtypeStruct((NPAD, 1), jnp.float32),
        ),
    )(oa, ob, b[None, :], W, a_src[:, None], a_dst[:, None])
    return h_pad, al_s[:, 0], al_d[:, 0]


def _tc_final_body(acc_ref, b_ref, out_ref):
    s = acc_ref[0] + acc_ref[1]
    o = s[:, :D] / (s[:, D:D + 1] + 1e-16) + b_ref[...]
    m = jnp.max(o, axis=1, keepdims=True)
    z = o - m
    out_ref[...] = z - jnp.log(jnp.sum(jnp.exp(z), axis=1, keepdims=True))


def _tc_final(acc, b):
    return pl.pallas_call(
        _tc_final_body,
        out_shape=jax.ShapeDtypeStruct((NPAD, D), jnp.float32),
    )(acc, b[None, :])


def _sc_edge_max(src, dst, asrc, adst):
    nt = asrc.shape[0]

    def body(src_h, dst_h, asrc_h, adst_h, e_h, m_h,
             asrc_v, adst_v, mx, src_c, dst_c, e_c, red_v, mseg, stage):
        cid = lax.axis_index("c")
        sid = lax.axis_index("s")
        wid = sid * NC + cid
        base = wid * EPW
        pltpu.sync_copy(asrc_h, asrc_v)
        pltpu.sync_copy(adst_h, adst_v)
        pltpu.sync_copy(src_h.at[pl.ds(base, EPW)], src_c)
        pltpu.sync_copy(dst_h.at[pl.ds(base, EPW)], dst_c)

        def zero_body(i, c):
            mx[pl.ds(i * L, L)] = jnp.full((L,), NEG, jnp.float32)
            return c
        lax.fori_loop(0, NPAD // L, zero_body, 0)

        def edge_body(j, c):
            s = src_c[pl.ds(j * L, L)]
            d = dst_c[pl.ds(j * L, L)]
            e = plsc.load_gather(asrc_v, [s]) + plsc.load_gather(adst_v, [d])
            e = jnp.where(e >= 0, e, 0.2 * e)
            e_c[pl.ds(j * L, L)] = e

            def cond(p):
                return jnp.max(jnp.where(p, 1, 0)) > 0

            def wbody(p):
                cur = plsc.load_gather(mx, [d])
                plsc.store_scatter(mx, [d], jnp.maximum(cur, e), mask=p)
                cur2 = plsc.load_gather(mx, [d])
                return jnp.logical_and(p, cur2 < e)

            lax.while_loop(cond, wbody, jnp.ones((L,), jnp.bool_))
            return c
        lax.fori_loop(0, EPW // L, edge_body, 0)
        pltpu.sync_copy(e_c, e_h.at[pl.ds(base, EPW)])

        pltpu.sync_copy(mx, stage.at[sid])
        plsc.subcore_barrier()
        pltpu.sync_copy(stage.at[:, pl.ds(sid * SEG, SEG)], red_v)

        def red_body(j, c):
            acc = red_v[0, pl.ds(j * L, L)]
            for t in range(1, NS):
                acc = jnp.maximum(acc, red_v[t, pl.ds(j * L, L)])
            mseg[pl.ds(j * L, L)] = acc
            return c
        lax.fori_loop(0, SEG // L, red_body, 0)
        pltpu.sync_copy(mseg, m_h.at[cid, pl.ds(sid * SEG, SEG)])

    fn = pl.kernel(
        body,
        out_type=(
            jax.ShapeDtypeStruct((E,), jnp.float32),
            jax.ShapeDtypeStruct((NC, NPAD), jnp.float32),
        ),
        mesh=_mesh(),
        scratch_types=[
            pltpu.VMEM((nt,), jnp.float32),
            pltpu.VMEM((nt,), jnp.float32),
            pltpu.VMEM((NPAD,), jnp.float32),
            pltpu.VMEM((EPW,), jnp.int32),
            pltpu.VMEM((EPW,), jnp.int32),
            pltpu.VMEM((EPW,), jnp.float32),
            pltpu.VMEM((NS, SEG), jnp.float32),
            pltpu.VMEM((SEG,), jnp.float32),
            pltpu.VMEM_SHARED((NS, NPAD), jnp.float32),
        ],
        name="sc_edge_max",
        compiler_params=pltpu.CompilerParams(needs_layout_passes=False),
    )
    return fn(src, dst, asrc, adst)


def _sc_aggregate(src, dst, e_edge, m2, h_pad):
    zeros_h = jnp.zeros((NPAD, DP), jnp.float32)

    def body(src_h, dst_h, e_h, m_h, h_h, z_h, out_h,
             m_v, mtmp, src_c, dst_c, e_c, sidx, didx, wv,
             rows0, rows1, out_sh, gsem0, gsem1, ssem0, ssem1):
        cid = lax.axis_index("c")
        sid = lax.axis_index("s")
        wid = sid * NC + cid
        base = wid * EPW
        rows = (rows0, rows1)
        gsems = (gsem0, gsem1)
        ssems = (ssem0, ssem1)

        pltpu.sync_copy(m_h.at[0], m_v)
        pltpu.sync_copy(src_h.at[pl.ds(base, EPW)], src_c)
        pltpu.sync_copy(dst_h.at[pl.ds(base, EPW)], dst_c)
        pltpu.sync_copy(e_h.at[pl.ds(base, EPW)], e_c)

        def max_chunk(k, c):
            pltpu.sync_copy(m_h.at[1, pl.ds(k * SEG, SEG)], mtmp)

            def max_body(j, cc):
                off = k * SEG + j * L
                m_v[pl.ds(off, L)] = jnp.maximum(
                    m_v[pl.ds(off, L)], mtmp[pl.ds(j * L, L)])
                return cc
            lax.fori_loop(0, SEG // L, max_body, 0)
            return c
        lax.fori_loop(0, NS, max_chunk, 0)

        pltpu.sync_copy(z_h.at[pl.ds(sid * SEG, SEG)],
                        out_sh.at[pl.ds(sid * SEG, SEG)])
        plsc.subcore_barrier()

        def prep(t, q):
            for j in range(K // L):
                off = t * K + j * L
                s = src_c[pl.ds(off, L)]
                dd = dst_c[pl.ds(off, L)]
                e = e_c[pl.ds(off, L)]
                mval = plsc.load_gather(m_v, [dd])
                w = jnp.exp(e - mval)
                sidx[q, pl.ds(j * L, L)] = s
                didx[q, pl.ds(j * L, L)] = dd
                wv[q, pl.ds(j * L, L)] = w

        def fire_gather(q):
            pltpu.async_copy(h_h.at[sidx.at[q]], rows[q], gsems[q])

        def wait_gather(q):
            pltpu.make_async_copy(h_h.at[sidx.at[q]], rows[q], gsems[q]).wait()

        def fire_scatter(q):
            pltpu.async_copy(rows[q], out_sh.at[didx.at[q]], ssems[q],
                             add=True)

        def wait_scatter(q):
            pltpu.make_async_copy(rows[q], out_sh.at[didx.at[q]],
                                  ssems[q]).wait()

        def scale(q):
            @plsc.parallel_loop(0, K, step=1, unroll=8)
            def scale_body(i):
                ridx = jnp.full((L,), 0, jnp.int32) + i
                wb = plsc.load_gather(wv.at[q], [ridx])
                for cc in range(DP // L):
                    cidx = lax.iota(jnp.int32, L) + cc * L
                    v = plsc.load_gather(rows[q], [ridx, cidx])
                    plsc.store_scatter(rows[q], [ridx, cidx], v * wb)

        pltpu.sync_copy(z_h.at[pl.ds(0, K)], rows1)
        for j in range(K // L):
            didx[1, pl.ds(j * L, L)] = jnp.zeros((L,), jnp.int32)
        fire_scatter(1)

        prep(0, 0)
        fire_gather(0)

        def half(s, p):
            wait_scatter(1 - p)
            prep(s + 1, 1 - p)
            fire_gather(1 - p)
            wait_gather(p)
            scale(p)
            fire_scatter(p)

        def stage_pair(i2, c):
            half(2 * i2, 0)
            half(2 * i2 + 1, 1)
            return c
        lax.fori_loop(0, (NSTAGES - 1) // 2, stage_pair, 0)
        wait_gather(0)
        scale(0)
        fire_scatter(0)
        wait_scatter(1)
        wait_scatter(0)

        plsc.subcore_barrier()
        pltpu.sync_copy(out_sh.at[pl.ds(sid * SEG, SEG)],
                        out_h.at[cid, pl.ds(sid * SEG, SEG)])

    fn = pl.kernel(
        body,
        out_type=jax.ShapeDtypeStruct((NC, NPAD, DP), jnp.float32),
        mesh=_mesh(),
        scratch_types=[
            pltpu.VMEM((NPAD,), jnp.float32),
            pltpu.VMEM((SEG,), jnp.float32),
            pltpu.VMEM((EPW,), jnp.int32),
            pltpu.VMEM((EPW,), jnp.int32),
            pltpu.VMEM((EPW,), jnp.float32),
            pltpu.VMEM((2, K), jnp.int32),
            pltpu.VMEM((2, K), jnp.int32),
            pltpu.VMEM((2, K), jnp.float32),
            pltpu.VMEM((K, DP), jnp.float32),
            pltpu.VMEM((K, DP), jnp.float32),
            pltpu.VMEM_SHARED((NPAD, DP), jnp.float32),
            pltpu.SemaphoreType.DMA,
            pltpu.SemaphoreType.DMA,
            pltpu.SemaphoreType.DMA,
            pltpu.SemaphoreType.DMA,
        ],
        name="sc_aggregate",
        compiler_params=pltpu.CompilerParams(
            needs_layout_passes=False, use_tc_tiling_on_sc=False),
    )
    return fn(src, dst, e_edge, m2, h_pad, zeros_h)


def kernel(x, edge_index, W1, a1_src, a1_dst, b1, W2, a2_src, a2_dst, b2):
    src = edge_index[0]
    dst = edge_index[1]
    ha1, hb1, as1, ad1 = _tc_first(x, W1, a1_src, a1_dst)
    e1, m1 = _sc_edge_max(src, dst, as1, ad1)
    o1a = _sc_aggregate(src, dst, e1, m1, ha1)
    o1b = _sc_aggregate(src, dst, e1, m1, hb1)
    h2p, as2, ad2 = _tc_mid(o1a, o1b, b1, W2, a2_src, a2_dst)
    e2, m2 = _sc_edge_max(src, dst, as2, ad2)
    o2 = _sc_aggregate(src, dst, e2, m2, h2p)
    out = _tc_final(o2, b2)
    return out[:N]


# speedup vs baseline: 2.1025x; 1.1230x over previous
import functools

import jax
import jax.numpy as jnp
from jax import lax
from jax.experimental import pallas as pl
from jax.experimental.pallas import tpu as pltpu
from jax.experimental.pallas import tpu_sc as plsc

N = 10000
E = 320000
NPAD = 10240
NC = 2
NS = 16
NW = NC * NS
EPW = E // NW
L = 16
K = 80
NSTAGES = EPW // K
SEG = NPAD // NS
D = 64
DP = D + 16
NEG = -1e30


def _mesh():
    return plsc.VectorSubcoreMesh(core_axis_name="c", subcore_axis_name="s")


def _pad_cols(h):
    n = h.shape[0]
    ones = jnp.ones((n, 1), jnp.float32)
    zeros = jnp.zeros((n, 15), jnp.float32)
    return jnp.concatenate([h, ones, zeros], axis=1)


def _tc_first_body(x_ref, w_ref, asrc_ref, adst_ref,
                   ha_ref, hb_ref, als_ref, ald_ref):
    h = jnp.dot(x_ref[...], w_ref[...], preferred_element_type=jnp.float32)
    ha_ref[...] = _pad_cols(h[:, :D])
    hb_ref[...] = _pad_cols(h[:, D:])
    als_ref[...] = h @ asrc_ref[...]
    ald_ref[...] = h @ adst_ref[...]


def _tc_first(x, W, a_src, a_dst):
    ha, hb, al_s, al_d = pl.pallas_call(
        _tc_first_body,
        out_shape=(
            jax.ShapeDtypeStruct((N, DP), jnp.float32),
            jax.ShapeDtypeStruct((N, DP), jnp.float32),
            jax.ShapeDtypeStruct((N, 1), jnp.float32),
            jax.ShapeDtypeStruct((N, 1), jnp.float32),
        ),
    )(x, W, a_src[:, None], a_dst[:, None])
    return ha, hb, al_s[:, 0], al_d[:, 0]


def _tc_mid_body(oa_ref, ob_ref, b_ref, w_ref, asrc_ref, adst_ref,
                 h_ref, als_ref, ald_ref):
    sa = oa_ref[0] + oa_ref[1]
    sb = ob_ref[0] + ob_ref[1]
    den = sa[:, D:D + 1] + 1e-16
    o = jnp.concatenate([sa[:, :D], sb[:, :D]], axis=1) / den + b_ref[...]
    x2 = jnp.where(o > 0, o, jnp.exp(jnp.minimum(o, 0.0)) - 1.0)
    h = jnp.dot(x2, w_ref[...], preferred_element_type=jnp.float32)
    h_ref[...] = _pad_cols(h)
    als_ref[...] = h @ asrc_ref[...]
    ald_ref[...] = h @ adst_ref[...]


def _tc_mid(oa, ob, b, W, a_src, a_dst):
    h_pad, al_s, al_d = pl.pallas_call(
        _tc_mid_body,
        out_shape=(
            jax.ShapeDtypeStruct((NPAD, DP), jnp.float32),
            jax.ShapeDtypeStruct((NPAD, 1), jnp.float32),
            jax.ShapeDtypeStruct((NPAD, 1), jnp.float32),
        ),
    )(oa, ob, b[None, :], W, a_src[:, None], a_dst[:, None])
    return h_pad, al_s[:, 0], al_d[:, 0]


def _tc_final_body(acc_ref, b_ref, out_ref):
    s = acc_ref[0] + acc_ref[1]
    o = s[:, :D] / (s[:, D:D + 1] + 1e-16) + b_ref[...]
    m = jnp.max(o, axis=1, keepdims=True)
    z = o - m
    out_ref[...] = z - jnp.log(jnp.sum(jnp.exp(z), axis=1, keepdims=True))


def _tc_final(acc, b):
    return pl.pallas_call(
        _tc_final_body,
        out_shape=jax.ShapeDtypeStruct((NPAD, D), jnp.float32),
    )(acc, b[None, :])


def _sc_edge_max(src, dst, asrc, adst):
    nt = asrc.shape[0]

    def body(src_h, dst_h, asrc_h, adst_h, e_h, m_h,
             asrc_v, adst_v, mx, src_c, dst_c, e_c, red_v, mseg, stage):
        cid = lax.axis_index("c")
        sid = lax.axis_index("s")
        wid = sid * NC + cid
        base = wid * EPW
        pltpu.sync_copy(asrc_h, asrc_v)
        pltpu.sync_copy(adst_h, adst_v)
        pltpu.sync_copy(src_h.at[pl.ds(base, EPW)], src_c)
        pltpu.sync_copy(dst_h.at[pl.ds(base, EPW)], dst_c)

        def zero_body(i, c):
            mx[pl.ds(i * L, L)] = jnp.full((L,), NEG, jnp.float32)
            return c
        lax.fori_loop(0, NPAD // L, zero_body, 0)

        @plsc.parallel_loop(0, EPW // L, step=1, unroll=8)
        def logit_body(j):
            s = src_c[pl.ds(j * L, L)]
            d = dst_c[pl.ds(j * L, L)]
            e = plsc.load_gather(asrc_v, [s]) + plsc.load_gather(adst_v, [d])
            e_c[pl.ds(j * L, L)] = jnp.where(e >= 0, e, 0.2 * e)

        def edge_body(j, c):
            d = dst_c[pl.ds(j * L, L)]
            e = e_c[pl.ds(j * L, L)]

            def cond(p):
                return jnp.any(p)

            def wbody(p):
                cur = plsc.load_gather(mx, [d])
                plsc.store_scatter(mx, [d], jnp.maximum(cur, e), mask=p)
                cur2 = plsc.load_gather(mx, [d])
                return jnp.logical_and(p, cur2 < e)

            lax.while_loop(cond, wbody, jnp.ones((L,), jnp.bool_))
            return c
        lax.fori_loop(0, EPW // L, edge_body, 0)
        pltpu.sync_copy(e_c, e_h.at[pl.ds(base, EPW)])

        pltpu.sync_copy(mx, stage.at[sid])
        plsc.subcore_barrier()
        pltpu.sync_copy(stage.at[:, pl.ds(sid * SEG, SEG)], red_v)

        def red_body(j, c):
            acc = red_v[0, pl.ds(j * L, L)]
            for t in range(1, NS):
                acc = jnp.maximum(acc, red_v[t, pl.ds(j * L, L)])
            mseg[pl.ds(j * L, L)] = acc
            return c
        lax.fori_loop(0, SEG // L, red_body, 0)
        pltpu.sync_copy(mseg, m_h.at[cid, pl.ds(sid * SEG, SEG)])

    fn = pl.kernel(
        body,
        out_type=(
            jax.ShapeDtypeStruct((E,), jnp.float32),
            jax.ShapeDtypeStruct((NC, NPAD), jnp.float32),
        ),
        mesh=_mesh(),
        scratch_types=[
            pltpu.VMEM((nt,), jnp.float32),
            pltpu.VMEM((nt,), jnp.float32),
            pltpu.VMEM((NPAD,), jnp.float32),
            pltpu.VMEM((EPW,), jnp.int32),
            pltpu.VMEM((EPW,), jnp.int32),
            pltpu.VMEM((EPW,), jnp.float32),
            pltpu.VMEM((NS, SEG), jnp.float32),
            pltpu.VMEM((SEG,), jnp.float32),
            pltpu.VMEM_SHARED((NS, NPAD), jnp.float32),
        ],
        name="sc_edge_max",
        compiler_params=pltpu.CompilerParams(needs_layout_passes=False),
    )
    return fn(src, dst, asrc, adst)


def _sc_aggregate(src, dst, e_edge, m2, h_pad):
    zeros_h = jnp.zeros((NPAD, DP), jnp.float32)

    def body(src_h, dst_h, e_h, m_h, h_h, z_h, out_h,
             m_v, mtmp, src_c, dst_c, e_c, sidx, didx, wv,
             rows0, rows1, out_sh, gsem0, gsem1, ssem0, ssem1):
        cid = lax.axis_index("c")
        sid = lax.axis_index("s")
        wid = sid * NC + cid
        base = wid * EPW
        rows = (rows0, rows1)
        gsems = (gsem0, gsem1)
        ssems = (ssem0, ssem1)

        pltpu.sync_copy(m_h.at[0], m_v)
        pltpu.sync_copy(src_h.at[pl.ds(base, EPW)], src_c)
        pltpu.sync_copy(dst_h.at[pl.ds(base, EPW)], dst_c)
        pltpu.sync_copy(e_h.at[pl.ds(base, EPW)], e_c)

        def max_chunk(k, c):
            pltpu.sync_copy(m_h.at[1, pl.ds(k * SEG, SEG)], mtmp)

            def max_body(j, cc):
                off = k * SEG + j * L
                m_v[pl.ds(off, L)] = jnp.maximum(
                    m_v[pl.ds(off, L)], mtmp[pl.ds(j * L, L)])
                return cc
            lax.fori_loop(0, SEG // L, max_body, 0)
            return c
        lax.fori_loop(0, NS, max_chunk, 0)

        pltpu.sync_copy(z_h.at[pl.ds(sid * SEG, SEG)],
                        out_sh.at[pl.ds(sid * SEG, SEG)])
        plsc.subcore_barrier()

        def prep(t, q):
            @plsc.parallel_loop(0, K // L, step=1, unroll=K // L)
            def prep_body(j):
                off = t * K + j * L
                s = src_c[pl.ds(off, L)]
                dd = dst_c[pl.ds(off, L)]
                e = e_c[pl.ds(off, L)]
                mval = plsc.load_gather(m_v, [dd])
                w = jnp.exp(e - mval)
                sidx[q, pl.ds(j * L, L)] = s
                didx[q, pl.ds(j * L, L)] = dd
                wv[q, pl.ds(j * L, L)] = w

        def fire_gather(q):
            pltpu.async_copy(h_h.at[sidx.at[q]], rows[q], gsems[q])

        def wait_gather(q):
            pltpu.make_async_copy(h_h.at[sidx.at[q]], rows[q], gsems[q]).wait()

        def fire_scatter(q):
            pltpu.async_copy(rows[q], out_sh.at[didx.at[q]], ssems[q],
                             add=True)

        def wait_scatter(q):
            pltpu.make_async_copy(rows[q], out_sh.at[didx.at[q]],
                                  ssems[q]).wait()

        def scale(q):
            @plsc.parallel_loop(0, K, step=1, unroll=16)
            def scale_body(i):
                ridx = jnp.full((L,), 0, jnp.int32) + i
                wb = plsc.load_gather(wv.at[q], [ridx])
                for cc in range(DP // L):
                    cidx = lax.iota(jnp.int32, L) + cc * L
                    v = plsc.load_gather(rows[q], [ridx, cidx])
                    plsc.store_scatter(rows[q], [ridx, cidx], v * wb)

        pltpu.sync_copy(z_h.at[pl.ds(0, K)], rows1)
        for j in range(K // L):
            didx[1, pl.ds(j * L, L)] = jnp.zeros((L,), jnp.int32)
        fire_scatter(1)

        prep(0, 0)
        fire_gather(0)

        def half(s, p):
            wait_scatter(1 - p)
            prep(s + 1, 1 - p)
            fire_gather(1 - p)
            wait_gather(p)
            scale(p)
            fire_scatter(p)

        def stage_pair(i2, c):
            half(2 * i2, 0)
            half(2 * i2 + 1, 1)
            return c
        lax.fori_loop(0, (NSTAGES - 1) // 2, stage_pair, 0)
        wait_gather(0)
        scale(0)
        fire_scatter(0)
        wait_scatter(1)
        wait_scatter(0)

        plsc.subcore_barrier()
        pltpu.sync_copy(out_sh.at[pl.ds(sid * SEG, SEG)],
                        out_h.at[cid, pl.ds(sid * SEG, SEG)])

    fn = pl.kernel(
        body,
        out_type=jax.ShapeDtypeStruct((NC, NPAD, DP), jnp.float32),
        mesh=_mesh(),
        scratch_types=[
            pltpu.VMEM((NPAD,), jnp.float32),
            pltpu.VMEM((SEG,), jnp.float32),
            pltpu.VMEM((EPW,), jnp.int32),
            pltpu.VMEM((EPW,), jnp.int32),
            pltpu.VMEM((EPW,), jnp.float32),
            pltpu.VMEM((2, K), jnp.int32),
            pltpu.VMEM((2, K), jnp.int32),
            pltpu.VMEM((2, K), jnp.float32),
            pltpu.VMEM((K, DP), jnp.float32),
            pltpu.VMEM((K, DP), jnp.float32),
            pltpu.VMEM_SHARED((NPAD, DP), jnp.float32),
            pltpu.SemaphoreType.DMA,
            pltpu.SemaphoreType.DMA,
            pltpu.SemaphoreType.DMA,
            pltpu.SemaphoreType.DMA,
        ],
        name="sc_aggregate",
        compiler_params=pltpu.CompilerParams(
            needs_layout_passes=False, use_tc_tiling_on_sc=False),
    )
    return fn(src, dst, e_edge, m2, h_pad, zeros_h)


def kernel(x, edge_index, W1, a1_src, a1_dst, b1, W2, a2_src, a2_dst, b2):
    src = edge_index[0]
    dst = edge_index[1]
    ha1, hb1, as1, ad1 = _tc_first(x, W1, a1_src, a1_dst)
    e1, m1 = _sc_edge_max(src, dst, as1, ad1)
    o1a = _sc_aggregate(src, dst, e1, m1, ha1)
    o1b = _sc_aggregate(src, dst, e1, m1, hb1)
    h2p, as2, ad2 = _tc_mid(o1a, o1b, b1, W2, a2_src, a2_dst)
    e2, m2 = _sc_edge_max(src, dst, as2, ad2)
    o2 = _sc_aggregate(src, dst, e2, m2, h2p)
    out = _tc_final(o2, b2)
    return out[:N]


# speedup vs baseline: 2.1979x; 1.0454x over previous
import functools

import jax
import jax.numpy as jnp
from jax import lax
from jax.experimental import pallas as pl
from jax.experimental.pallas import tpu as pltpu
from jax.experimental.pallas import tpu_sc as plsc

N = 10000
E = 320000
NPAD = 10240
NC = 2
NS = 16
NW = NC * NS
EPW = E // NW
L = 16
K = 80
NSTAGES = EPW // K
SEG = NPAD // NS
D = 64
DP = D + 16
NEG = -1e30


def _mesh():
    return plsc.VectorSubcoreMesh(core_axis_name="c", subcore_axis_name="s")


def _pad_cols(h):
    n = h.shape[0]
    ones = jnp.ones((n, 1), jnp.float32)
    zeros = jnp.zeros((n, 15), jnp.float32)
    return jnp.concatenate([h, ones, zeros], axis=1)


def _tc_first_body(x_ref, w_ref, asrc_ref, adst_ref,
                   ha_ref, hb_ref, als_ref, ald_ref):
    h = jnp.dot(x_ref[...], w_ref[...], preferred_element_type=jnp.float32)
    ha_ref[...] = _pad_cols(h[:, :D])
    hb_ref[...] = h[:, D:]
    als_ref[...] = h @ asrc_ref[...]
    ald_ref[...] = h @ adst_ref[...]


def _tc_first(x, W, a_src, a_dst):
    ha, hb, al_s, al_d = pl.pallas_call(
        _tc_first_body,
        out_shape=(
            jax.ShapeDtypeStruct((N, DP), jnp.float32),
            jax.ShapeDtypeStruct((N, D), jnp.float32),
            jax.ShapeDtypeStruct((N, 1), jnp.float32),
            jax.ShapeDtypeStruct((N, 1), jnp.float32),
        ),
    )(x, W, a_src[:, None], a_dst[:, None])
    return ha, hb, al_s[:, 0], al_d[:, 0]


def _tc_mid_body(oa_ref, ob_ref, b_ref, w_ref, asrc_ref, adst_ref,
                 h_ref, als_ref, ald_ref):
    sa = oa_ref[0] + oa_ref[1]
    sb = ob_ref[0] + ob_ref[1]
    den = sa[:, D:D + 1] + 1e-16
    o = jnp.concatenate([sa[:, :D], sb], axis=1) / den + b_ref[...]
    x2 = jnp.where(o > 0, o, jnp.exp(jnp.minimum(o, 0.0)) - 1.0)
    h = jnp.dot(x2, w_ref[...], preferred_element_type=jnp.float32)
    h_ref[...] = _pad_cols(h)
    als_ref[...] = h @ asrc_ref[...]
    ald_ref[...] = h @ adst_ref[...]


def _tc_mid(oa, ob, b, W, a_src, a_dst):
    h_pad, al_s, al_d = pl.pallas_call(
        _tc_mid_body,
        out_shape=(
            jax.ShapeDtypeStruct((NPAD, DP), jnp.float32),
            jax.ShapeDtypeStruct((NPAD, 1), jnp.float32),
            jax.ShapeDtypeStruct((NPAD, 1), jnp.float32),
        ),
    )(oa, ob, b[None, :], W, a_src[:, None], a_dst[:, None])
    return h_pad, al_s[:, 0], al_d[:, 0]


def _tc_final_body(acc_ref, b_ref, out_ref):
    s = acc_ref[0] + acc_ref[1]
    o = s[:, :D] / (s[:, D:D + 1] + 1e-16) + b_ref[...]
    m = jnp.max(o, axis=1, keepdims=True)
    z = o - m
    out_ref[...] = z - jnp.log(jnp.sum(jnp.exp(z), axis=1, keepdims=True))


def _tc_final(acc, b):
    return pl.pallas_call(
        _tc_final_body,
        out_shape=jax.ShapeDtypeStruct((NPAD, D), jnp.float32),
    )(acc, b[None, :])


def _sc_edge_max(src, dst, asrc, adst):
    nt = asrc.shape[0]

    def body(src_h, dst_h, asrc_h, adst_h, e_h, m_h,
             asrc_v, adst_v, mx, src_c, dst_c, e_c, red_v, mseg, stage):
        cid = lax.axis_index("c")
        sid = lax.axis_index("s")
        wid = sid * NC + cid
        base = wid * EPW
        pltpu.sync_copy(asrc_h, asrc_v)
        pltpu.sync_copy(adst_h, adst_v)
        pltpu.sync_copy(src_h.at[pl.ds(base, EPW)], src_c)
        pltpu.sync_copy(dst_h.at[pl.ds(base, EPW)], dst_c)

        @plsc.parallel_loop(0, NPAD // L, step=1, unroll=8)
        def zero_body(i):
            mx[pl.ds(i * L, L)] = jnp.full((L,), NEG, jnp.float32)

        @plsc.parallel_loop(0, EPW // L, step=1, unroll=8)
        def logit_body(j):
            s = src_c[pl.ds(j * L, L)]
            d = dst_c[pl.ds(j * L, L)]
            e = plsc.load_gather(asrc_v, [s]) + plsc.load_gather(adst_v, [d])
            e_c[pl.ds(j * L, L)] = jnp.where(e >= 0, e, 0.2 * e)

        def edge_body(j, c):
            d = dst_c[pl.ds(j * L, L)]
            e = e_c[pl.ds(j * L, L)]

            def cond(p):
                return jnp.any(p)

            def wbody(p):
                cur = plsc.load_gather(mx, [d])
                plsc.store_scatter(mx, [d], jnp.maximum(cur, e), mask=p)
                cur2 = plsc.load_gather(mx, [d])
                return jnp.logical_and(p, cur2 < e)

            lax.while_loop(cond, wbody, jnp.ones((L,), jnp.bool_))
            return c
        lax.fori_loop(0, EPW // L, edge_body, 0)
        pltpu.sync_copy(e_c, e_h.at[pl.ds(base, EPW)])

        pltpu.sync_copy(mx, stage.at[sid])
        plsc.subcore_barrier()
        pltpu.sync_copy(stage.at[:, pl.ds(sid * SEG, SEG)], red_v)

        @plsc.parallel_loop(0, SEG // L, step=1, unroll=4)
        def red_body(j):
            acc = red_v[0, pl.ds(j * L, L)]
            for t in range(1, NS):
                acc = jnp.maximum(acc, red_v[t, pl.ds(j * L, L)])
            mseg[pl.ds(j * L, L)] = acc
        pltpu.sync_copy(mseg, m_h.at[cid, pl.ds(sid * SEG, SEG)])

    fn = pl.kernel(
        body,
        out_type=(
            jax.ShapeDtypeStruct((E,), jnp.float32),
            jax.ShapeDtypeStruct((NC, NPAD), jnp.float32),
        ),
        mesh=_mesh(),
        scratch_types=[
            pltpu.VMEM((nt,), jnp.float32),
            pltpu.VMEM((nt,), jnp.float32),
            pltpu.VMEM((NPAD,), jnp.float32),
            pltpu.VMEM((EPW,), jnp.int32),
            pltpu.VMEM((EPW,), jnp.int32),
            pltpu.VMEM((EPW,), jnp.float32),
            pltpu.VMEM((NS, SEG), jnp.float32),
            pltpu.VMEM((SEG,), jnp.float32),
            pltpu.VMEM_SHARED((NS, NPAD), jnp.float32),
        ],
        name="sc_edge_max",
        compiler_params=pltpu.CompilerParams(needs_layout_passes=False),
    )
    return fn(src, dst, asrc, adst)


def _sc_aggregate(src, dst, e_edge, m2, h_pad, with_den):
    dp = DP if with_den else D
    zeros_h = jnp.zeros((NPAD, dp), jnp.float32)

    def body(src_h, dst_h, e_h, m_h, h_h, z_h, out_h,
             m_v, mtmp, src_c, dst_c, e_c, sidx, didx, wv,
             rows0, rows1, out_sh, gsem0, gsem1, ssem0, ssem1):
        cid = lax.axis_index("c")
        sid = lax.axis_index("s")
        wid = sid * NC + cid
        base = wid * EPW
        rows = (rows0, rows1)
        gsems = (gsem0, gsem1)
        ssems = (ssem0, ssem1)

        pltpu.sync_copy(m_h.at[0], m_v)
        pltpu.sync_copy(src_h.at[pl.ds(base, EPW)], src_c)
        pltpu.sync_copy(dst_h.at[pl.ds(base, EPW)], dst_c)
        pltpu.sync_copy(e_h.at[pl.ds(base, EPW)], e_c)

        def max_chunk(k, c):
            pltpu.sync_copy(m_h.at[1, pl.ds(k * SEG, SEG)], mtmp)

            @plsc.parallel_loop(0, SEG // L, step=1, unroll=8)
            def max_body(j):
                off = k * SEG + j * L
                m_v[pl.ds(off, L)] = jnp.maximum(
                    m_v[pl.ds(off, L)], mtmp[pl.ds(j * L, L)])
            return c
        lax.fori_loop(0, NS, max_chunk, 0)

        pltpu.sync_copy(z_h.at[pl.ds(sid * SEG, SEG)],
                        out_sh.at[pl.ds(sid * SEG, SEG)])
        plsc.subcore_barrier()

        def prep(t, q):
            @plsc.parallel_loop(0, K // L, step=1, unroll=K // L)
            def prep_body(j):
                off = t * K + j * L
                s = src_c[pl.ds(off, L)]
                dd = dst_c[pl.ds(off, L)]
                e = e_c[pl.ds(off, L)]
                mval = plsc.load_gather(m_v, [dd])
                w = jnp.exp(e - mval)
                sidx[q, pl.ds(j * L, L)] = s
                didx[q, pl.ds(j * L, L)] = dd
                wv[q, pl.ds(j * L, L)] = w

        def fire_gather(q):
            pltpu.async_copy(h_h.at[sidx.at[q]], rows[q], gsems[q])

        def wait_gather(q):
            pltpu.make_async_copy(h_h.at[sidx.at[q]], rows[q], gsems[q]).wait()

        def fire_scatter(q):
            pltpu.async_copy(rows[q], out_sh.at[didx.at[q]], ssems[q],
                             add=True)

        def wait_scatter(q):
            pltpu.make_async_copy(rows[q], out_sh.at[didx.at[q]],
                                  ssems[q]).wait()

        def scale(q):
            @plsc.parallel_loop(0, K, step=1, unroll=16)
            def scale_body(i):
                ridx = jnp.full((L,), 0, jnp.int32) + i
                wb = plsc.load_gather(wv.at[q], [ridx])
                for cc in range(dp // L):
                    cidx = lax.iota(jnp.int32, L) + cc * L
                    v = plsc.load_gather(rows[q], [ridx, cidx])
                    plsc.store_scatter(rows[q], [ridx, cidx], v * wb)

        pltpu.sync_copy(z_h.at[pl.ds(0, K)], rows1)
        for j in range(K // L):
            didx[1, pl.ds(j * L, L)] = jnp.zeros((L,), jnp.int32)
        fire_scatter(1)

        prep(0, 0)
        fire_gather(0)

        def half(s, p):
            wait_scatter(1 - p)
            prep(s + 1, 1 - p)
            fire_gather(1 - p)
            wait_gather(p)
            scale(p)
            fire_scatter(p)

        def stage_pair(i2, c):
            half(2 * i2, 0)
            half(2 * i2 + 1, 1)
            return c
        lax.fori_loop(0, (NSTAGES - 1) // 2, stage_pair, 0)
        wait_gather(0)
        scale(0)
        fire_scatter(0)
        wait_scatter(1)
        wait_scatter(0)

        plsc.subcore_barrier()
        pltpu.sync_copy(out_sh.at[pl.ds(sid * SEG, SEG)],
                        out_h.at[cid, pl.ds(sid * SEG, SEG)])

    fn = pl.kernel(
        body,
        out_type=jax.ShapeDtypeStruct((NC, NPAD, dp), jnp.float32),
        mesh=_mesh(),
        scratch_types=[
            pltpu.VMEM((NPAD,), jnp.float32),
            pltpu.VMEM((SEG,), jnp.float32),
            pltpu.VMEM((EPW,), jnp.int32),
            pltpu.VMEM((EPW,), jnp.int32),
            pltpu.VMEM((EPW,), jnp.float32),
            pltpu.VMEM((2, K), jnp.int32),
            pltpu.VMEM((2, K), jnp.int32),
            pltpu.VMEM((2, K), jnp.float32),
            pltpu.VMEM((K, dp), jnp.float32),
            pltpu.VMEM((K, dp), jnp.float32),
            pltpu.VMEM_SHARED((NPAD, dp), jnp.float32),
            pltpu.SemaphoreType.DMA,
            pltpu.SemaphoreType.DMA,
            pltpu.SemaphoreType.DMA,
            pltpu.SemaphoreType.DMA,
        ],
        name="sc_aggregate",
        compiler_params=pltpu.CompilerParams(
            needs_layout_passes=False, use_tc_tiling_on_sc=False),
    )
    return fn(src, dst, e_edge, m2, h_pad, zeros_h)


def kernel(x, edge_index, W1, a1_src, a1_dst, b1, W2, a2_src, a2_dst, b2):
    src = edge_index[0]
    dst = edge_index[1]
    ha1, hb1, as1, ad1 = _tc_first(x, W1, a1_src, a1_dst)
    e1, m1 = _sc_edge_max(src, dst, as1, ad1)
    o1a = _sc_aggregate(src, dst, e1, m1, ha1, True)
    o1b = _sc_aggregate(src, dst, e1, m1, hb1, False)
    h2p, as2, ad2 = _tc_mid(o1a, o1b, b1, W2, a2_src, a2_dst)
    e2, m2 = _sc_edge_max(src, dst, as2, ad2)
    o2 = _sc_aggregate(src, dst, e2, m2, h2p, True)
    out = _tc_final(o2, b2)
    return out[:N]
